# Initial kernel scaffold; baseline (speedup 1.0000x reference)
import jax
import jax.numpy as jnp
from jax.experimental import pallas as pl


def kernel(input_tensor):
    raise NotImplementedError("write your pallas kernel here")


# speedup vs baseline: 3.6615x; 3.6615x over previous
import functools

import jax
import jax.numpy as jnp
from jax import lax
from jax.experimental import pallas as pl
from jax.experimental.pallas import tpu as pltpu
from jax.experimental.pallas import tpu_sc as plsc

N_ROWS = 16384
N_COLS = 4096
TOTAL = N_ROWS * N_COLS
NUM_CORES = 2
NUM_SUBCORES = 16
NW = NUM_CORES * NUM_SUBCORES
PER_W = TOTAL // NW
LANES = 16
PERIOD = 128
CHUNK = 8 * N_COLS
N_CHUNKS = PER_W // CHUNK


def _apply_acts(buf):
    def body(p, carry):
        base = p * PERIOD
        for v in range(4):
            s = base + v * LANES
            x = buf[pl.ds(s, LANES)]
            buf[pl.ds(s, LANES)] = jnp.maximum(x, 0.0)
        for v in range(4):
            s = base + 64 + v * LANES
            x = buf[pl.ds(s, LANES)]
            buf[pl.ds(s, LANES)] = x / (1.0 + jnp.exp(-x))
        return carry

    lax.fori_loop(0, CHUNK // PERIOD, body, 0)


_MESH = plsc.VectorSubcoreMesh(core_axis_name="c", subcore_axis_name="s")


@functools.partial(
    pl.kernel,
    mesh=_MESH,
    out_type=jax.ShapeDtypeStruct((TOTAL,), jnp.float32),
    scratch_types=[pltpu.VMEM((CHUNK,), jnp.float32)],
)
def _mixed_act_sc(x_hbm, out_hbm, buf):
    wid = lax.axis_index("s") * NUM_CORES + lax.axis_index("c")
    base = wid * PER_W

    def step(i, carry):
        off = pl.multiple_of(base + i * CHUNK, CHUNK)
        pltpu.sync_copy(x_hbm.at[pl.ds(off, CHUNK)], buf)
        _apply_acts(buf)
        pltpu.sync_copy(buf, out_hbm.at[pl.ds(off, CHUNK)])
        return carry

    lax.fori_loop(0, N_CHUNKS, step, 0)


def kernel(input_tensor):
    flat = input_tensor.reshape(TOTAL)
    out = _mixed_act_sc(flat)
    return out.reshape(N_ROWS, N_COLS)


# speedup vs baseline: 5.5864x; 1.5257x over previous
import functools

import jax
import jax.numpy as jnp
from jax import lax
from jax.experimental import pallas as pl
from jax.experimental.pallas import tpu as pltpu
from jax.experimental.pallas import tpu_sc as plsc

N_ROWS = 16384
N_COLS = 4096
TOTAL = N_ROWS * N_COLS
NUM_CORES = 2
NUM_SUBCORES = 16
NW = NUM_CORES * NUM_SUBCORES
PER_W = TOTAL // NW
LANES = 16
PERIOD = 128
CHUNK = 4 * N_COLS
N_CHUNKS = PER_W // CHUNK
N_GROUPS = N_CHUNKS // 2


def _apply_acts(src, dst):
    def body(p, carry):
        base = p * PERIOD
        for v in range(4):
            s = base + v * LANES
            x = src[pl.ds(s, LANES)]
            dst[pl.ds(s, LANES)] = jnp.maximum(x, 0.0)
        for v in range(4):
            s = base + 64 + v * LANES
            x = src[pl.ds(s, LANES)]
            dst[pl.ds(s, LANES)] = x / (1.0 + jnp.exp(-x))
        return carry

    lax.fori_loop(0, CHUNK // PERIOD, body, 0)


_MESH = plsc.VectorSubcoreMesh(core_axis_name="c", subcore_axis_name="s")


@functools.partial(
    pl.kernel,
    mesh=_MESH,
    out_type=jax.ShapeDtypeStruct((TOTAL,), jnp.float32),
    scratch_types=[
        pltpu.VMEM((CHUNK,), jnp.float32),
        pltpu.VMEM((CHUNK,), jnp.float32),
        pltpu.VMEM((CHUNK,), jnp.float32),
        pltpu.VMEM((CHUNK,), jnp.float32),
        pltpu.SemaphoreType.DMA,
        pltpu.SemaphoreType.DMA,
        pltpu.SemaphoreType.DMA,
        pltpu.SemaphoreType.DMA,
    ],
)
def _mixed_act_sc(x_hbm, out_hbm, ib0, ib1, ob0, ob1, is0, is1, os0, os1):
    wid = lax.axis_index("s") * NUM_CORES + lax.axis_index("c")
    base = wid * PER_W

    def _off(i):
        return pl.multiple_of(base + i * CHUNK, 8)

    def _src(i):
        return x_hbm.at[pl.ds(_off(i), CHUNK)]

    def _dst(i):
        return out_hbm.at[pl.ds(_off(i), CHUNK)]

    pltpu.async_copy(_src(0), ib0, is0)
    pltpu.async_copy(_src(1), ib1, is1)

    def group(g, carry):
        for b, (ib, ob, isem, osem) in enumerate(
            ((ib0, ob0, is0, os0), (ib1, ob1, is1, os1))
        ):
            i = 2 * g + b
            pltpu.make_async_copy(_src(i), ib, isem).wait()
            @pl.when(g > 0)
            def _wait_store():
                pltpu.make_async_copy(ob, _dst(i), osem).wait()

            _apply_acts(ib, ob)
            pltpu.async_copy(ob, _dst(i), osem)

            @pl.when(g < N_GROUPS - 1)
            def _next_load():
                pltpu.async_copy(_src(i + 2), ib, isem)

        return carry

    lax.fori_loop(0, N_GROUPS, group, 0)

    pltpu.make_async_copy(ob0, _dst(N_CHUNKS - 2), os0).wait()
    pltpu.make_async_copy(ob1, _dst(N_CHUNKS - 1), os1).wait()


def kernel(input_tensor):
    flat = input_tensor.reshape(TOTAL)
    out = _mixed_act_sc(flat)
    return out.reshape(N_ROWS, N_COLS)


# speedup vs baseline: 17.3928x; 3.1134x over previous
import functools

import jax
import jax.numpy as jnp
from jax import lax
from jax.experimental import pallas as pl
from jax.experimental.pallas import tpu as pltpu
from jax.experimental.pallas import tpu_sc as plsc

N_ROWS = 16384
N_COLS = 4096
NUM_CORES = 2
NUM_SUBCORES = 16
NW = NUM_CORES * NUM_SUBCORES
ROWS_PER_W = N_ROWS // NW
LANES = 16
PERIOD = 128
CHUNK_ROWS = 4
N_CHUNKS = ROWS_PER_W // CHUNK_ROWS
N_GROUPS = N_CHUNKS // 2


def _apply_acts(src, dst):
    for r in range(CHUNK_ROWS):

        def body(q, carry, r=r):
            base = q * PERIOD
            for v in range(4):
                s = base + v * LANES
                x = src[r, pl.ds(s, LANES)]
                dst[r, pl.ds(s, LANES)] = jnp.maximum(x, 0.0)
            for v in range(4):
                s = base + 64 + v * LANES
                x = src[r, pl.ds(s, LANES)]
                dst[r, pl.ds(s, LANES)] = x / (1.0 + jnp.exp(-x))
            return carry

        lax.fori_loop(0, N_COLS // PERIOD, body, 0)


_MESH = plsc.VectorSubcoreMesh(core_axis_name="c", subcore_axis_name="s")


@functools.partial(
    pl.kernel,
    mesh=_MESH,
    out_type=jax.ShapeDtypeStruct((N_ROWS, N_COLS), jnp.float32),
    scratch_types=[
        pltpu.VMEM((CHUNK_ROWS, N_COLS), jnp.float32),
        pltpu.VMEM((CHUNK_ROWS, N_COLS), jnp.float32),
        pltpu.VMEM((CHUNK_ROWS, N_COLS), jnp.float32),
        pltpu.VMEM((CHUNK_ROWS, N_COLS), jnp.float32),
        pltpu.SemaphoreType.DMA,
        pltpu.SemaphoreType.DMA,
        pltpu.SemaphoreType.DMA,
        pltpu.SemaphoreType.DMA,
    ],
)
def _mixed_act_sc(x_hbm, out_hbm, ib0, ib1, ob0, ob1, is0, is1, os0, os1):
    wid = lax.axis_index("s") * NUM_CORES + lax.axis_index("c")
    base_row = wid * ROWS_PER_W

    def _src(i):
        return x_hbm.at[pl.ds(base_row + i * CHUNK_ROWS, CHUNK_ROWS), :]

    def _dst(i):
        return out_hbm.at[pl.ds(base_row + i * CHUNK_ROWS, CHUNK_ROWS), :]

    pltpu.async_copy(_src(0), ib0, is0)
    pltpu.async_copy(_src(1), ib1, is1)

    def group(g, carry):
        for b, (ib, ob, isem, osem) in enumerate(
            ((ib0, ob0, is0, os0), (ib1, ob1, is1, os1))
        ):
            i = 2 * g + b
            pltpu.make_async_copy(_src(i), ib, isem).wait()
            @pl.when(g > 0)
            def _wait_store():
                pltpu.make_async_copy(ob, _dst(i), osem).wait()

            _apply_acts(ib, ob)
            pltpu.async_copy(ob, _dst(i), osem)

            @pl.when(g < N_GROUPS - 1)
            def _next_load():
                pltpu.async_copy(_src(i + 2), ib, isem)

        return carry

    lax.fori_loop(0, N_GROUPS, group, 0)

    pltpu.make_async_copy(ob0, _dst(N_CHUNKS - 2), os0).wait()
    pltpu.make_async_copy(ob1, _dst(N_CHUNKS - 1), os1).wait()


def kernel(input_tensor):
    return _mixed_act_sc(input_tensor)
